# Initial kernel scaffold; baseline (speedup 1.0000x reference)
import jax
import jax.numpy as jnp
from jax.experimental import pallas as pl


def kernel(x, edge_index, W1, att_src1, att_dst1, b1, W2, att_src2, att_dst2, b2, Wf, bf):
    raise NotImplementedError("write your pallas kernel here")


# speedup vs baseline: 26.0000x; 26.0000x over previous
import functools

import jax
import jax.numpy as jnp
from jax import lax
from jax.experimental import pallas as pl
from jax.experimental.pallas import tpu as pltpu
from jax.experimental.pallas import tpu_sc as plsc

N = 10000
NP = 10240
D_IN = 128
DH = 32
CHUNK = 2048
NCHUNK = 82
E_HALF = CHUNK * NCHUNK
E_PAD = 2 * E_HALF
A_SLICE = E_HALF // 16
A_VECS = A_SLICE // 16
B_VECS = CHUNK // 16
NZV = NP // 16
BLK = 2048

_mesh = plsc.VectorSubcoreMesh(core_axis_name="c", subcore_axis_name="s")


@functools.partial(
    pl.kernel,
    out_type=(
        jax.ShapeDtypeStruct((2, 16, NP), jnp.float32),
        jax.ShapeDtypeStruct((2, DH, NP), jnp.float32),
        jax.ShapeDtypeStruct((E_PAD,), jnp.float32),
    ),
    mesh=_mesh,
    compiler_params=pltpu.CompilerParams(needs_layout_passes=False),
    scratch_types=[
        pltpu.VMEM((NP,), jnp.float32),
        pltpu.VMEM((NP,), jnp.float32),
        pltpu.VMEM((NP,), jnp.float32),
        pltpu.VMEM((NP,), jnp.float32),
        pltpu.VMEM((NP,), jnp.float32),
        pltpu.VMEM((A_SLICE,), jnp.int32),
        pltpu.VMEM((A_SLICE,), jnp.int32),
        pltpu.VMEM((A_SLICE,), jnp.float32),
    ],
)
def _gat_aggregate(src_hbm, dst_hbm, a_s_hbm, a_d_hbm, hT_hbm,
                   denom_out, acc_out, w_hbm,
                   tabA, tabB, acc0, acc1, denom, srcb, dstb, wb):
    c = lax.axis_index("c")
    s = lax.axis_index("s")

    zeros = jnp.zeros((16,), jnp.float32)

    def zero_body(i, carry):
        sl = pl.ds(i * 16, 16)
        acc0[sl] = zeros
        acc1[sl] = zeros
        denom[sl] = zeros
        return carry

    lax.fori_loop(0, NZV, zero_body, 0)

    pltpu.sync_copy(a_s_hbm, tabA)
    pltpu.sync_copy(a_d_hbm, tabB)
    baseA = c * E_HALF + s * A_SLICE
    pltpu.sync_copy(src_hbm.at[pl.ds(baseA, A_SLICE)], srcb)
    pltpu.sync_copy(dst_hbm.at[pl.ds(baseA, A_SLICE)], dstb)

    def wbody(i, carry):
        sl = pl.ds(i * 16, 16)
        sv = srcb[sl]
        dv = dstb[sl]
        e = plsc.load_gather(tabA, [sv]) + plsc.load_gather(tabB, [dv])
        e = jnp.where(e >= 0.0, e, e * 0.2)
        wv = jnp.exp(e)
        wb[sl] = wv
        plsc.addupdate_scatter(denom, [dv], wv)
        return carry

    lax.fori_loop(0, A_VECS, wbody, 0)

    pltpu.sync_copy(wb, w_hbm.at[pl.ds(baseA, A_SLICE)])
    pltpu.sync_copy(denom, denom_out.at[c, s])
    plsc.subcore_barrier()

    pltpu.sync_copy(hT_hbm.at[s], tabA)
    pltpu.sync_copy(hT_hbm.at[s + 16], tabB)
    baseB = c * E_HALF

    def chunk_body(g, carry):
        off = baseB + g * CHUNK
        pltpu.sync_copy(src_hbm.at[pl.ds(off, CHUNK)], srcb.at[pl.ds(0, CHUNK)])
        pltpu.sync_copy(dst_hbm.at[pl.ds(off, CHUNK)], dstb.at[pl.ds(0, CHUNK)])
        pltpu.sync_copy(w_hbm.at[pl.ds(off, CHUNK)], wb.at[pl.ds(0, CHUNK)])

        def vbody(i, icarry):
            sl = pl.ds(i * 16, 16)
            sv = srcb[sl]
            dv = dstb[sl]
            wv = wb[sl]
            plsc.addupdate_scatter(acc0, [dv], plsc.load_gather(tabA, [sv]) * wv)
            plsc.addupdate_scatter(acc1, [dv], plsc.load_gather(tabB, [sv]) * wv)
            return icarry

        lax.fori_loop(0, B_VECS, vbody, 0)
        return carry

    lax.fori_loop(0, NCHUNK, chunk_body, 0)

    pltpu.sync_copy(acc0, acc_out.at[c, s])
    pltpu.sync_copy(acc1, acc_out.at[c, s + 16])


def _proj_body(xT_ref, WT_ref, att_s_ref, att_d_ref, hT_ref, as_ref, ad_ref):
    h = jnp.dot(WT_ref[...], xT_ref[...], preferred_element_type=jnp.float32)
    hT_ref[...] = h
    as_ref[...] = jnp.dot(att_s_ref[...], h, preferred_element_type=jnp.float32)
    ad_ref[...] = jnp.dot(att_d_ref[...], h, preferred_element_type=jnp.float32)


def _proj(xT, WT, att_s, att_d, d_in):
    return pl.pallas_call(
        _proj_body,
        grid=(NP // BLK,),
        in_specs=[
            pl.BlockSpec((d_in, BLK), lambda j: (0, j)),
            pl.BlockSpec((DH, d_in), lambda j: (0, 0)),
            pl.BlockSpec((1, DH), lambda j: (0, 0)),
            pl.BlockSpec((1, DH), lambda j: (0, 0)),
        ],
        out_specs=[
            pl.BlockSpec((DH, BLK), lambda j: (0, j)),
            pl.BlockSpec((1, BLK), lambda j: (0, j)),
            pl.BlockSpec((1, BLK), lambda j: (0, j)),
        ],
        out_shape=[
            jax.ShapeDtypeStruct((DH, NP), jnp.float32),
            jax.ShapeDtypeStruct((1, NP), jnp.float32),
            jax.ShapeDtypeStruct((1, NP), jnp.float32),
        ],
    )(xT, WT, att_s, att_d)


def _norm_proj_body(acc_ref, den_ref, b_ref, WT_ref, att_s_ref, att_d_ref,
                    hT_ref, as_ref, ad_ref):
    accs = acc_ref[0] + acc_ref[1]
    den = jnp.sum(den_ref[0] + den_ref[1], axis=0, keepdims=True)
    h = jnp.maximum(accs / (den + 1e-16) + b_ref[...], 0.0)
    hT2 = jnp.dot(WT_ref[...], h, preferred_element_type=jnp.float32)
    hT_ref[...] = hT2
    as_ref[...] = jnp.dot(att_s_ref[...], hT2, preferred_element_type=jnp.float32)
    ad_ref[...] = jnp.dot(att_d_ref[...], hT2, preferred_element_type=jnp.float32)


def _norm_proj(acc, den, b, WT, att_s, att_d):
    return pl.pallas_call(
        _norm_proj_body,
        grid=(NP // BLK,),
        in_specs=[
            pl.BlockSpec((2, DH, BLK), lambda j: (0, 0, j)),
            pl.BlockSpec((2, 16, BLK), lambda j: (0, 0, j)),
            pl.BlockSpec((DH, 1), lambda j: (0, 0)),
            pl.BlockSpec((DH, DH), lambda j: (0, 0)),
            pl.BlockSpec((1, DH), lambda j: (0, 0)),
            pl.BlockSpec((1, DH), lambda j: (0, 0)),
        ],
        out_specs=[
            pl.BlockSpec((DH, BLK), lambda j: (0, j)),
            pl.BlockSpec((1, BLK), lambda j: (0, j)),
            pl.BlockSpec((1, BLK), lambda j: (0, j)),
        ],
        out_shape=[
            jax.ShapeDtypeStruct((DH, NP), jnp.float32),
            jax.ShapeDtypeStruct((1, NP), jnp.float32),
            jax.ShapeDtypeStruct((1, NP), jnp.float32),
        ],
    )(acc, den, b, WT, att_s, att_d)


def _final_body(acc_ref, den_ref, b_ref, Wf_ref, bf_ref, out_ref):
    accs = acc_ref[0] + acc_ref[1]
    den = jnp.sum(den_ref[0] + den_ref[1], axis=0, keepdims=True)
    h = jnp.maximum(accs / (den + 1e-16) + b_ref[...], 0.0)
    out_ref[...] = lax.dot_general(
        h, Wf_ref[...], (((0,), (0,)), ((), ())),
        preferred_element_type=jnp.float32) + bf_ref[...]


def _final(acc, den, b, Wf, bf):
    return pl.pallas_call(
        _final_body,
        grid=(NP // BLK,),
        in_specs=[
            pl.BlockSpec((2, DH, BLK), lambda j: (0, 0, j)),
            pl.BlockSpec((2, 16, BLK), lambda j: (0, 0, j)),
            pl.BlockSpec((DH, 1), lambda j: (0, 0)),
            pl.BlockSpec((DH, DH), lambda j: (0, 0)),
            pl.BlockSpec((1, DH), lambda j: (0, 0)),
        ],
        out_specs=pl.BlockSpec((BLK, DH), lambda j: (j, 0)),
        out_shape=jax.ShapeDtypeStruct((NP, DH), jnp.float32),
    )(acc, den, b, Wf, bf)


def kernel(x, edge_index, W1, att_src1, att_dst1, b1,
           W2, att_src2, att_dst2, b2, Wf, bf):
    n_edges = edge_index.shape[1]
    loop = jnp.arange(N, dtype=jnp.int32)
    npad = E_PAD - (n_edges + N)
    src = jnp.concatenate(
        [edge_index[0].astype(jnp.int32), loop, jnp.zeros((npad,), jnp.int32)])
    dst = jnp.concatenate(
        [edge_index[1].astype(jnp.int32), loop, jnp.full((npad,), N, jnp.int32)])

    xT = jnp.pad(x, ((0, NP - N), (0, 0))).T

    hT1, as1, ad1 = _proj(xT, W1.T, att_src1.reshape(1, DH),
                          att_dst1.reshape(1, DH), D_IN)
    den1, acc1, _ = _gat_aggregate(src, dst, as1.reshape(NP), ad1.reshape(NP), hT1)

    hT2, as2, ad2 = _norm_proj(acc1, den1, b1.reshape(DH, 1), W2.T,
                               att_src2.reshape(1, DH), att_dst2.reshape(1, DH))
    den2, acc2, _ = _gat_aggregate(src, dst, as2.reshape(NP), ad2.reshape(NP), hT2)

    out = _final(acc2, den2, b2.reshape(DH, 1), Wf, bf.reshape(1, DH))
    return out[:N]


# speedup vs baseline: 39.0199x; 1.5008x over previous
import functools

import jax
import jax.numpy as jnp
from jax import lax
from jax.experimental import pallas as pl
from jax.experimental.pallas import tpu as pltpu
from jax.experimental.pallas import tpu_sc as plsc

N = 10000
NP = 10240
D_IN = 128
DH = 32
CHUNK = 4096
NCHUNK = 42
E_HALF = CHUNK * NCHUNK
E_PAD = 2 * E_HALF
A_SLICE = E_HALF // 16
A_VECS = A_SLICE // 16
B_VECS = CHUNK // 16
NZV = NP // 16
BLK = 2048

_mesh = plsc.VectorSubcoreMesh(core_axis_name="c", subcore_axis_name="s")


@functools.partial(
    pl.kernel,
    out_type=(
        jax.ShapeDtypeStruct((2, 16, NP), jnp.float32),
        jax.ShapeDtypeStruct((2, DH, NP), jnp.float32),
        jax.ShapeDtypeStruct((E_PAD,), jnp.float32),
    ),
    mesh=_mesh,
    compiler_params=pltpu.CompilerParams(needs_layout_passes=False),
    scratch_types=[
        pltpu.VMEM((NP,), jnp.float32),
        pltpu.VMEM((NP,), jnp.float32),
        pltpu.VMEM((NP,), jnp.float32),
        pltpu.VMEM((NP,), jnp.float32),
        pltpu.VMEM((NP,), jnp.float32),
        pltpu.VMEM((A_SLICE,), jnp.int32),
        pltpu.VMEM((A_SLICE,), jnp.int32),
        pltpu.VMEM((A_SLICE,), jnp.float32),
        pltpu.SemaphoreType.DMA,
        pltpu.SemaphoreType.DMA,
    ],
)
def _gat_aggregate(src_hbm, dst_hbm, a_s_hbm, a_d_hbm, hT_hbm,
                   denom_out, acc_out, w_hbm,
                   tabA, tabB, acc0, acc1, denom, srcb, dstb, wb,
                   sem0, sem1):
    c = lax.axis_index("c")
    s = lax.axis_index("s")

    zeros = jnp.zeros((16,), jnp.float32)

    baseA = c * E_HALF + s * A_SLICE
    cp_as = pltpu.async_copy(a_s_hbm, tabA, sem0)
    cp_ad = pltpu.async_copy(a_d_hbm, tabB, sem0)
    cp_src = pltpu.async_copy(src_hbm.at[pl.ds(baseA, A_SLICE)], srcb, sem0)
    cp_dst = pltpu.async_copy(dst_hbm.at[pl.ds(baseA, A_SLICE)], dstb, sem0)

    def zero_body(i, carry):
        sl = pl.ds(i * 16, 16)
        acc0[sl] = zeros
        acc1[sl] = zeros
        denom[sl] = zeros
        return carry

    lax.fori_loop(0, NZV, zero_body, 0)

    cp_as.wait()
    cp_ad.wait()
    cp_src.wait()
    cp_dst.wait()

    def wbody(i, carry):
        base = i * 64
        for u in range(4):
            sl = pl.ds(base + u * 16, 16)
            sv = srcb[sl]
            dv = dstb[sl]
            e = plsc.load_gather(tabA, [sv]) + plsc.load_gather(tabB, [dv])
            e = jnp.where(e >= 0.0, e, e * 0.2)
            wv = jnp.exp(e)
            wb[sl] = wv
            plsc.addupdate_scatter(denom, [dv], wv)
        return carry

    lax.fori_loop(0, A_VECS // 4, wbody, 0)

    pltpu.sync_copy(wb, w_hbm.at[pl.ds(baseA, A_SLICE)])
    pltpu.sync_copy(denom, denom_out.at[c, s])
    plsc.subcore_barrier()

    cp_h0 = pltpu.async_copy(hT_hbm.at[s], tabA, sem0)
    cp_h1 = pltpu.async_copy(hT_hbm.at[s + 16], tabB, sem0)
    baseB = c * E_HALF
    sems = (sem0, sem1)

    def _issue(g, b):
        off = baseB + g * CHUNK
        slot = pl.ds(b * CHUNK, CHUNK)
        pltpu.async_copy(src_hbm.at[pl.ds(off, CHUNK)], srcb.at[slot], sems[b])
        pltpu.async_copy(dst_hbm.at[pl.ds(off, CHUNK)], dstb.at[slot], sems[b])
        pltpu.async_copy(w_hbm.at[pl.ds(off, CHUNK)], wb.at[slot], sems[b])

    def _wait(g, b):
        off = baseB + g * CHUNK
        slot = pl.ds(b * CHUNK, CHUNK)
        pltpu.make_async_copy(src_hbm.at[pl.ds(off, CHUNK)], srcb.at[slot], sems[b]).wait()
        pltpu.make_async_copy(dst_hbm.at[pl.ds(off, CHUNK)], dstb.at[slot], sems[b]).wait()
        pltpu.make_async_copy(w_hbm.at[pl.ds(off, CHUNK)], wb.at[slot], sems[b]).wait()

    def _compute(b):
        sbase = b * CHUNK

        def vbody(i, icarry):
            vb = sbase + i * 64
            for u in range(4):
                sl = pl.ds(vb + u * 16, 16)
                sv = srcb[sl]
                dv = dstb[sl]
                wv = wb[sl]
                plsc.addupdate_scatter(acc0, [dv], plsc.load_gather(tabA, [sv]) * wv)
                plsc.addupdate_scatter(acc1, [dv], plsc.load_gather(tabB, [sv]) * wv)
            return icarry

        lax.fori_loop(0, B_VECS // 4, vbody, 0)

    cp_h0.wait()
    cp_h1.wait()
    _issue(0, 0)
    _issue(1, 1)

    def mbody(g2, carry):
        g = g2 * 2
        _wait(g, 0)
        _compute(0)
        _issue(g + 2, 0)
        _wait(g + 1, 1)
        _compute(1)
        _issue(g + 3, 1)
        return carry

    lax.fori_loop(0, NCHUNK // 2 - 1, mbody, 0)

    _wait(NCHUNK - 2, 0)
    _compute(0)
    _wait(NCHUNK - 1, 1)
    _compute(1)

    pltpu.sync_copy(acc0, acc_out.at[c, s])
    pltpu.sync_copy(acc1, acc_out.at[c, s + 16])


def _proj_body(xT_ref, WT_ref, att_s_ref, att_d_ref, hT_ref, as_ref, ad_ref):
    h = jnp.dot(WT_ref[...], xT_ref[...], preferred_element_type=jnp.float32)
    hT_ref[...] = h
    as_ref[...] = jnp.dot(att_s_ref[...], h, preferred_element_type=jnp.float32)
    ad_ref[...] = jnp.dot(att_d_ref[...], h, preferred_element_type=jnp.float32)


def _proj(xT, WT, att_s, att_d, d_in):
    return pl.pallas_call(
        _proj_body,
        grid=(NP // BLK,),
        in_specs=[
            pl.BlockSpec((d_in, BLK), lambda j: (0, j)),
            pl.BlockSpec((DH, d_in), lambda j: (0, 0)),
            pl.BlockSpec((1, DH), lambda j: (0, 0)),
            pl.BlockSpec((1, DH), lambda j: (0, 0)),
        ],
        out_specs=[
            pl.BlockSpec((DH, BLK), lambda j: (0, j)),
            pl.BlockSpec((1, BLK), lambda j: (0, j)),
            pl.BlockSpec((1, BLK), lambda j: (0, j)),
        ],
        out_shape=[
            jax.ShapeDtypeStruct((DH, NP), jnp.float32),
            jax.ShapeDtypeStruct((1, NP), jnp.float32),
            jax.ShapeDtypeStruct((1, NP), jnp.float32),
        ],
    )(xT, WT, att_s, att_d)


def _norm_proj_body(acc_ref, den_ref, b_ref, WT_ref, att_s_ref, att_d_ref,
                    hT_ref, as_ref, ad_ref):
    accs = acc_ref[0] + acc_ref[1]
    den = jnp.sum(den_ref[0] + den_ref[1], axis=0, keepdims=True)
    h = jnp.maximum(accs / (den + 1e-16) + b_ref[...], 0.0)
    hT2 = jnp.dot(WT_ref[...], h, preferred_element_type=jnp.float32)
    hT_ref[...] = hT2
    as_ref[...] = jnp.dot(att_s_ref[...], hT2, preferred_element_type=jnp.float32)
    ad_ref[...] = jnp.dot(att_d_ref[...], hT2, preferred_element_type=jnp.float32)


def _norm_proj(acc, den, b, WT, att_s, att_d):
    return pl.pallas_call(
        _norm_proj_body,
        grid=(NP // BLK,),
        in_specs=[
            pl.BlockSpec((2, DH, BLK), lambda j: (0, 0, j)),
            pl.BlockSpec((2, 16, BLK), lambda j: (0, 0, j)),
            pl.BlockSpec((DH, 1), lambda j: (0, 0)),
            pl.BlockSpec((DH, DH), lambda j: (0, 0)),
            pl.BlockSpec((1, DH), lambda j: (0, 0)),
            pl.BlockSpec((1, DH), lambda j: (0, 0)),
        ],
        out_specs=[
            pl.BlockSpec((DH, BLK), lambda j: (0, j)),
            pl.BlockSpec((1, BLK), lambda j: (0, j)),
            pl.BlockSpec((1, BLK), lambda j: (0, j)),
        ],
        out_shape=[
            jax.ShapeDtypeStruct((DH, NP), jnp.float32),
            jax.ShapeDtypeStruct((1, NP), jnp.float32),
            jax.ShapeDtypeStruct((1, NP), jnp.float32),
        ],
    )(acc, den, b, WT, att_s, att_d)


def _final_body(acc_ref, den_ref, b_ref, Wf_ref, bf_ref, out_ref):
    accs = acc_ref[0] + acc_ref[1]
    den = jnp.sum(den_ref[0] + den_ref[1], axis=0, keepdims=True)
    h = jnp.maximum(accs / (den + 1e-16) + b_ref[...], 0.0)
    out_ref[...] = lax.dot_general(
        h, Wf_ref[...], (((0,), (0,)), ((), ())),
        preferred_element_type=jnp.float32) + bf_ref[...]


def _final(acc, den, b, Wf, bf):
    return pl.pallas_call(
        _final_body,
        grid=(NP // BLK,),
        in_specs=[
            pl.BlockSpec((2, DH, BLK), lambda j: (0, 0, j)),
            pl.BlockSpec((2, 16, BLK), lambda j: (0, 0, j)),
            pl.BlockSpec((DH, 1), lambda j: (0, 0)),
            pl.BlockSpec((DH, DH), lambda j: (0, 0)),
            pl.BlockSpec((1, DH), lambda j: (0, 0)),
        ],
        out_specs=pl.BlockSpec((BLK, DH), lambda j: (j, 0)),
        out_shape=jax.ShapeDtypeStruct((NP, DH), jnp.float32),
    )(acc, den, b, Wf, bf)


def kernel(x, edge_index, W1, att_src1, att_dst1, b1,
           W2, att_src2, att_dst2, b2, Wf, bf):
    n_edges = edge_index.shape[1]
    loop = jnp.arange(N, dtype=jnp.int32)
    npad = E_PAD - (n_edges + N)
    src = jnp.concatenate(
        [edge_index[0].astype(jnp.int32), loop, jnp.zeros((npad,), jnp.int32)])
    dst = jnp.concatenate(
        [edge_index[1].astype(jnp.int32), loop, jnp.full((npad,), N, jnp.int32)])

    xT = jnp.pad(x, ((0, NP - N), (0, 0))).T

    hT1, as1, ad1 = _proj(xT, W1.T, att_src1.reshape(1, DH),
                          att_dst1.reshape(1, DH), D_IN)
    den1, acc1, _ = _gat_aggregate(src, dst, as1.reshape(NP), ad1.reshape(NP), hT1)

    hT2, as2, ad2 = _norm_proj(acc1, den1, b1.reshape(DH, 1), W2.T,
                               att_src2.reshape(1, DH), att_dst2.reshape(1, DH))
    den2, acc2, _ = _gat_aggregate(src, dst, as2.reshape(NP), ad2.reshape(NP), hT2)

    out = _final(acc2, den2, b2.reshape(DH, 1), Wf, bf.reshape(1, DH))
    return out[:N]


# speedup vs baseline: 67.5585x; 1.7314x over previous
import functools

import jax
import jax.numpy as jnp
from jax import lax
from jax.experimental import pallas as pl
from jax.experimental.pallas import tpu as pltpu
from jax.experimental.pallas import tpu_sc as plsc

N = 10000
NP = 10240
D_IN = 128
DH = 32
CHUNK = 4096
NCHUNK = 42
E_HALF = CHUNK * NCHUNK
E_PAD = 2 * E_HALF
A_SLICE = E_HALF // 16
A_VECS = A_SLICE // 16
B_VECS = CHUNK // 16
NZV = NP // 16
BLK = 2048

_mesh = plsc.VectorSubcoreMesh(core_axis_name="c", subcore_axis_name="s")


@functools.partial(
    pl.kernel,
    out_type=(
        jax.ShapeDtypeStruct((2, 16, NP), jnp.float32),
        jax.ShapeDtypeStruct((2, DH, NP), jnp.float32),
        jax.ShapeDtypeStruct((E_PAD,), jnp.float32),
    ),
    mesh=_mesh,
    compiler_params=pltpu.CompilerParams(needs_layout_passes=False),
    scratch_types=[
        pltpu.VMEM((NP,), jnp.float32),
        pltpu.VMEM((NP,), jnp.float32),
        pltpu.VMEM((NP,), jnp.float32),
        pltpu.VMEM((NP,), jnp.float32),
        pltpu.VMEM((NP,), jnp.float32),
        pltpu.VMEM((A_SLICE,), jnp.int32),
        pltpu.VMEM((A_SLICE,), jnp.int32),
        pltpu.VMEM((A_SLICE,), jnp.float32),
        pltpu.SemaphoreType.DMA,
        pltpu.SemaphoreType.DMA,
    ],
)
def _gat_aggregate(src_hbm, dst_hbm, a_s_hbm, a_d_hbm, hT_hbm,
                   denom_out, acc_out, w_hbm,
                   tabA, tabB, acc0, acc1, denom, srcb, dstb, wb,
                   sem0, sem1):
    c = lax.axis_index("c")
    s = lax.axis_index("s")

    zeros = jnp.zeros((16,), jnp.float32)

    baseA = c * E_HALF + s * A_SLICE
    cp_as = pltpu.async_copy(a_s_hbm, tabA, sem0)
    cp_ad = pltpu.async_copy(a_d_hbm, tabB, sem0)
    cp_src = pltpu.async_copy(src_hbm.at[pl.ds(baseA, A_SLICE)], srcb, sem0)
    cp_dst = pltpu.async_copy(dst_hbm.at[pl.ds(baseA, A_SLICE)], dstb, sem0)

    def zero_body(i, carry):
        sl = pl.ds(i * 16, 16)
        acc0[sl] = zeros
        acc1[sl] = zeros
        denom[sl] = zeros
        return carry

    lax.fori_loop(0, NZV, zero_body, 0)

    cp_as.wait()
    cp_ad.wait()
    cp_src.wait()
    cp_dst.wait()

    def wbody(i, carry):
        base = i * 64
        sls = [pl.ds(base + u * 16, 16) for u in range(4)]
        svs = [srcb[sl] for sl in sls]
        dvs = [dstb[sl] for sl in sls]
        gas = [plsc.load_gather(tabA, [sv]) for sv in svs]
        gds = [plsc.load_gather(tabB, [dv]) for dv in dvs]
        es = [ga + gd for ga, gd in zip(gas, gds)]
        es = [jnp.where(e >= 0.0, e, e * 0.2) for e in es]
        ws = [jnp.exp(e) for e in es]
        for sl, wv in zip(sls, ws):
            wb[sl] = wv
        for dv, wv in zip(dvs, ws):
            plsc.addupdate_scatter(denom, [dv], wv)
        return carry

    lax.fori_loop(0, A_VECS // 4, wbody, 0)

    pltpu.sync_copy(wb, w_hbm.at[pl.ds(baseA, A_SLICE)])
    pltpu.sync_copy(denom, denom_out.at[c, s])
    plsc.subcore_barrier()

    cp_h0 = pltpu.async_copy(hT_hbm.at[s], tabA, sem0)
    cp_h1 = pltpu.async_copy(hT_hbm.at[s + 16], tabB, sem0)
    baseB = c * E_HALF
    sems = (sem0, sem1)

    def _issue(g, b):
        off = baseB + g * CHUNK
        slot = pl.ds(b * CHUNK, CHUNK)
        pltpu.async_copy(src_hbm.at[pl.ds(off, CHUNK)], srcb.at[slot], sems[b])
        pltpu.async_copy(dst_hbm.at[pl.ds(off, CHUNK)], dstb.at[slot], sems[b])
        pltpu.async_copy(w_hbm.at[pl.ds(off, CHUNK)], wb.at[slot], sems[b])

    def _wait(g, b):
        off = baseB + g * CHUNK
        slot = pl.ds(b * CHUNK, CHUNK)
        pltpu.make_async_copy(src_hbm.at[pl.ds(off, CHUNK)], srcb.at[slot], sems[b]).wait()
        pltpu.make_async_copy(dst_hbm.at[pl.ds(off, CHUNK)], dstb.at[slot], sems[b]).wait()
        pltpu.make_async_copy(w_hbm.at[pl.ds(off, CHUNK)], wb.at[slot], sems[b]).wait()

    def _compute(b):
        sbase = b * CHUNK

        def vbody(i, icarry):
            vb = sbase + i * 64
            sls = [pl.ds(vb + u * 16, 16) for u in range(4)]
            svs = [srcb[sl] for sl in sls]
            dvs = [dstb[sl] for sl in sls]
            wvs = [wb[sl] for sl in sls]
            g0s = [plsc.load_gather(tabA, [sv]) for sv in svs]
            g1s = [plsc.load_gather(tabB, [sv]) for sv in svs]
            m0s = [g * w for g, w in zip(g0s, wvs)]
            m1s = [g * w for g, w in zip(g1s, wvs)]
            for dv, m in zip(dvs, m0s):
                plsc.addupdate_scatter(acc0, [dv], m)
            for dv, m in zip(dvs, m1s):
                plsc.addupdate_scatter(acc1, [dv], m)
            return icarry

        lax.fori_loop(0, B_VECS // 4, vbody, 0)

    cp_h0.wait()
    cp_h1.wait()
    _issue(0, 0)
    _issue(1, 1)

    def mbody(g2, carry):
        g = g2 * 2
        _wait(g, 0)
        _compute(0)
        _issue(g + 2, 0)
        _wait(g + 1, 1)
        _compute(1)
        _issue(g + 3, 1)
        return carry

    lax.fori_loop(0, NCHUNK // 2 - 1, mbody, 0)

    _wait(NCHUNK - 2, 0)
    _compute(0)
    _wait(NCHUNK - 1, 1)
    _compute(1)

    pltpu.sync_copy(acc0, acc_out.at[c, s])
    pltpu.sync_copy(acc1, acc_out.at[c, s + 16])


def _proj_body(xT_ref, WT_ref, att_s_ref, att_d_ref, hT_ref, as_ref, ad_ref):
    h = jnp.dot(WT_ref[...], xT_ref[...], preferred_element_type=jnp.float32)
    hT_ref[...] = h
    as_ref[...] = jnp.dot(att_s_ref[...], h, preferred_element_type=jnp.float32)
    ad_ref[...] = jnp.dot(att_d_ref[...], h, preferred_element_type=jnp.float32)


def _proj(xT, WT, att_s, att_d, d_in):
    return pl.pallas_call(
        _proj_body,
        grid=(NP // BLK,),
        in_specs=[
            pl.BlockSpec((d_in, BLK), lambda j: (0, j)),
            pl.BlockSpec((DH, d_in), lambda j: (0, 0)),
            pl.BlockSpec((1, DH), lambda j: (0, 0)),
            pl.BlockSpec((1, DH), lambda j: (0, 0)),
        ],
        out_specs=[
            pl.BlockSpec((DH, BLK), lambda j: (0, j)),
            pl.BlockSpec((1, BLK), lambda j: (0, j)),
            pl.BlockSpec((1, BLK), lambda j: (0, j)),
        ],
        out_shape=[
            jax.ShapeDtypeStruct((DH, NP), jnp.float32),
            jax.ShapeDtypeStruct((1, NP), jnp.float32),
            jax.ShapeDtypeStruct((1, NP), jnp.float32),
        ],
    )(xT, WT, att_s, att_d)


def _norm_proj_body(acc_ref, den_ref, b_ref, WT_ref, att_s_ref, att_d_ref,
                    hT_ref, as_ref, ad_ref):
    accs = acc_ref[0] + acc_ref[1]
    den = jnp.sum(den_ref[0] + den_ref[1], axis=0, keepdims=True)
    h = jnp.maximum(accs / (den + 1e-16) + b_ref[...], 0.0)
    hT2 = jnp.dot(WT_ref[...], h, preferred_element_type=jnp.float32)
    hT_ref[...] = hT2
    as_ref[...] = jnp.dot(att_s_ref[...], hT2, preferred_element_type=jnp.float32)
    ad_ref[...] = jnp.dot(att_d_ref[...], hT2, preferred_element_type=jnp.float32)


def _norm_proj(acc, den, b, WT, att_s, att_d):
    return pl.pallas_call(
        _norm_proj_body,
        grid=(NP // BLK,),
        in_specs=[
            pl.BlockSpec((2, DH, BLK), lambda j: (0, 0, j)),
            pl.BlockSpec((2, 16, BLK), lambda j: (0, 0, j)),
            pl.BlockSpec((DH, 1), lambda j: (0, 0)),
            pl.BlockSpec((DH, DH), lambda j: (0, 0)),
            pl.BlockSpec((1, DH), lambda j: (0, 0)),
            pl.BlockSpec((1, DH), lambda j: (0, 0)),
        ],
        out_specs=[
            pl.BlockSpec((DH, BLK), lambda j: (0, j)),
            pl.BlockSpec((1, BLK), lambda j: (0, j)),
            pl.BlockSpec((1, BLK), lambda j: (0, j)),
        ],
        out_shape=[
            jax.ShapeDtypeStruct((DH, NP), jnp.float32),
            jax.ShapeDtypeStruct((1, NP), jnp.float32),
            jax.ShapeDtypeStruct((1, NP), jnp.float32),
        ],
    )(acc, den, b, WT, att_s, att_d)


def _final_body(acc_ref, den_ref, b_ref, Wf_ref, bf_ref, out_ref):
    accs = acc_ref[0] + acc_ref[1]
    den = jnp.sum(den_ref[0] + den_ref[1], axis=0, keepdims=True)
    h = jnp.maximum(accs / (den + 1e-16) + b_ref[...], 0.0)
    out_ref[...] = lax.dot_general(
        h, Wf_ref[...], (((0,), (0,)), ((), ())),
        preferred_element_type=jnp.float32) + bf_ref[...]


def _final(acc, den, b, Wf, bf):
    return pl.pallas_call(
        _final_body,
        grid=(NP // BLK,),
        in_specs=[
            pl.BlockSpec((2, DH, BLK), lambda j: (0, 0, j)),
            pl.BlockSpec((2, 16, BLK), lambda j: (0, 0, j)),
            pl.BlockSpec((DH, 1), lambda j: (0, 0)),
            pl.BlockSpec((DH, DH), lambda j: (0, 0)),
            pl.BlockSpec((1, DH), lambda j: (0, 0)),
        ],
        out_specs=pl.BlockSpec((BLK, DH), lambda j: (j, 0)),
        out_shape=jax.ShapeDtypeStruct((NP, DH), jnp.float32),
    )(acc, den, b, Wf, bf)


def kernel(x, edge_index, W1, att_src1, att_dst1, b1,
           W2, att_src2, att_dst2, b2, Wf, bf):
    n_edges = edge_index.shape[1]
    loop = jnp.arange(N, dtype=jnp.int32)
    npad = E_PAD - (n_edges + N)
    src = jnp.concatenate(
        [edge_index[0].astype(jnp.int32), loop, jnp.zeros((npad,), jnp.int32)])
    dst = jnp.concatenate(
        [edge_index[1].astype(jnp.int32), loop, jnp.full((npad,), N, jnp.int32)])

    xT = jnp.pad(x, ((0, NP - N), (0, 0))).T

    hT1, as1, ad1 = _proj(xT, W1.T, att_src1.reshape(1, DH),
                          att_dst1.reshape(1, DH), D_IN)
    den1, acc1, _ = _gat_aggregate(src, dst, as1.reshape(NP), ad1.reshape(NP), hT1)

    hT2, as2, ad2 = _norm_proj(acc1, den1, b1.reshape(DH, 1), W2.T,
                               att_src2.reshape(1, DH), att_dst2.reshape(1, DH))
    den2, acc2, _ = _gat_aggregate(src, dst, as2.reshape(NP), ad2.reshape(NP), hT2)

    out = _final(acc2, den2, b2.reshape(DH, 1), Wf, bf.reshape(1, DH))
    return out[:N]


# speedup vs baseline: 72.1292x; 1.0677x over previous
import functools

import jax
import jax.numpy as jnp
from jax import lax
from jax.experimental import pallas as pl
from jax.experimental.pallas import tpu as pltpu
from jax.experimental.pallas import tpu_sc as plsc

N = 10000
NP = 10240
D_IN = 128
DH = 32
CHUNK = 3200
NCHUNK = 52
E_HALF = CHUNK * NCHUNK
E_PAD = 2 * E_HALF
V_ITERS = CHUNK // 64
NZV = NP // 16
BLK = 2048
RBLK = 2048

_mesh = plsc.VectorSubcoreMesh(core_axis_name="c", subcore_axis_name="s")


@functools.partial(
    pl.kernel,
    out_type=(
        jax.ShapeDtypeStruct((2, NP), jnp.float32),
        jax.ShapeDtypeStruct((2, DH, NP), jnp.float32),
    ),
    mesh=_mesh,
    compiler_params=pltpu.CompilerParams(needs_layout_passes=False),
    scratch_types=[
        pltpu.VMEM((NP,), jnp.float32),
        pltpu.VMEM((NP,), jnp.float32),
        pltpu.VMEM((NP,), jnp.float32),
        pltpu.VMEM((NP,), jnp.float32),
        pltpu.VMEM((NP,), jnp.float32),
        pltpu.VMEM((NP,), jnp.float32),
        pltpu.VMEM((NP,), jnp.float32),
        pltpu.VMEM((2 * CHUNK,), jnp.int32),
        pltpu.VMEM((2 * CHUNK,), jnp.int32),
        pltpu.SemaphoreType.DMA,
        pltpu.SemaphoreType.DMA,
    ],
)
def _gat_aggregate(src_hbm, dst_hbm, a_s_hbm, a_d_hbm, hT_hbm,
                   denom_out, acc_out,
                   tabAS, tabAD, tabH0, tabH1, acc0, acc1, denom,
                   srcb, dstb, sem0, sem1):
    c = lax.axis_index("c")
    s = lax.axis_index("s")
    L = hT_hbm.shape[1]

    cp0 = pltpu.async_copy(a_s_hbm.at[0], tabAS.at[pl.ds(0, L)], sem0)
    cp1 = pltpu.async_copy(a_d_hbm.at[0], tabAD.at[pl.ds(0, L)], sem0)
    cp2 = pltpu.async_copy(hT_hbm.at[s], tabH0.at[pl.ds(0, L)], sem0)
    cp3 = pltpu.async_copy(hT_hbm.at[s + 16], tabH1.at[pl.ds(0, L)], sem0)

    zeros = jnp.zeros((16,), jnp.float32)

    def zero_body(i, carry):
        sl = pl.ds(i * 16, 16)
        acc0[sl] = zeros
        acc1[sl] = zeros
        denom[sl] = zeros
        return carry

    lax.fori_loop(0, NZV, zero_body, 0)

    cp0.wait()
    cp1.wait()
    cp2.wait()
    cp3.wait()

    base = c * E_HALF
    sems = (sem0, sem1)

    def _issue(g, b):
        off = base + g * CHUNK
        slot = pl.ds(b * CHUNK, CHUNK)
        pltpu.async_copy(src_hbm.at[pl.ds(off, CHUNK)], srcb.at[slot], sems[b])
        pltpu.async_copy(dst_hbm.at[pl.ds(off, CHUNK)], dstb.at[slot], sems[b])

    def _wait(g, b):
        off = base + g * CHUNK
        slot = pl.ds(b * CHUNK, CHUNK)
        pltpu.make_async_copy(src_hbm.at[pl.ds(off, CHUNK)], srcb.at[slot], sems[b]).wait()
        pltpu.make_async_copy(dst_hbm.at[pl.ds(off, CHUNK)], dstb.at[slot], sems[b]).wait()

    def _compute(b):
        sbase = b * CHUNK

        def vbody(i, icarry):
            vb = sbase + i * 64
            sls = [pl.ds(vb + u * 16, 16) for u in range(4)]
            svs = [srcb[sl] for sl in sls]
            dvs = [dstb[sl] for sl in sls]
            gas = [plsc.load_gather(tabAS, [sv]) for sv in svs]
            gds = [plsc.load_gather(tabAD, [dv]) for dv in dvs]
            h0s = [plsc.load_gather(tabH0, [sv]) for sv in svs]
            h1s = [plsc.load_gather(tabH1, [sv]) for sv in svs]
            es = [ga + gd for ga, gd in zip(gas, gds)]
            es = [jnp.where(e >= 0.0, e, e * 0.2) for e in es]
            ws = [jnp.exp(e) for e in es]
            m0s = [h * w for h, w in zip(h0s, ws)]
            m1s = [h * w for h, w in zip(h1s, ws)]
            for dv, w in zip(dvs, ws):
                plsc.addupdate_scatter(denom, [dv], w)
            for dv, m in zip(dvs, m0s):
                plsc.addupdate_scatter(acc0, [dv], m)
            for dv, m in zip(dvs, m1s):
                plsc.addupdate_scatter(acc1, [dv], m)
            return icarry

        lax.fori_loop(0, V_ITERS, vbody, 0)

    _issue(0, 0)
    _issue(1, 1)

    def mbody(g2, carry):
        g = g2 * 2
        _wait(g, 0)
        _compute(0)
        _issue(g + 2, 0)
        _wait(g + 1, 1)
        _compute(1)
        _issue(g + 3, 1)
        return carry

    lax.fori_loop(0, NCHUNK // 2 - 1, mbody, 0)

    _wait(NCHUNK - 2, 0)
    _compute(0)
    _wait(NCHUNK - 1, 1)
    _compute(1)

    pltpu.sync_copy(acc0, acc_out.at[c, s])
    pltpu.sync_copy(acc1, acc_out.at[c, s + 16])

    @pl.when(s == 0)
    def _():
        pltpu.sync_copy(denom, denom_out.at[c])


def _proj_body(x_ref, W_ref, att_s_ref, att_d_ref, hT_ref, as_ref, ad_ref):
    hT = lax.dot_general(W_ref[...], x_ref[...], (((0,), (1,)), ((), ())),
                         preferred_element_type=jnp.float32)
    hT_ref[...] = hT
    as_ref[...] = jnp.dot(att_s_ref[...], hT, preferred_element_type=jnp.float32)
    ad_ref[...] = jnp.dot(att_d_ref[...], hT, preferred_element_type=jnp.float32)


def _proj(x, W, att_s, att_d):
    return pl.pallas_call(
        _proj_body,
        grid=(pl.cdiv(N, RBLK),),
        in_specs=[
            pl.BlockSpec((RBLK, D_IN), lambda j: (j, 0)),
            pl.BlockSpec((D_IN, DH), lambda j: (0, 0)),
            pl.BlockSpec((1, DH), lambda j: (0, 0)),
            pl.BlockSpec((1, DH), lambda j: (0, 0)),
        ],
        out_specs=[
            pl.BlockSpec((DH, RBLK), lambda j: (0, j)),
            pl.BlockSpec((1, RBLK), lambda j: (0, j)),
            pl.BlockSpec((1, RBLK), lambda j: (0, j)),
        ],
        out_shape=[
            jax.ShapeDtypeStruct((DH, NP), jnp.float32),
            jax.ShapeDtypeStruct((1, NP), jnp.float32),
            jax.ShapeDtypeStruct((1, NP), jnp.float32),
        ],
    )(x, W, att_s, att_d)


def _norm_proj_body(acc_ref, den_ref, b_ref, WT_ref, att_s_ref, att_d_ref,
                    hT_ref, as_ref, ad_ref):
    accs = acc_ref[0] + acc_ref[1]
    den = jnp.sum(den_ref[...], axis=0, keepdims=True)
    h = jnp.maximum(accs / (den + 1e-16) + b_ref[...], 0.0)
    hT2 = jnp.dot(WT_ref[...], h, preferred_element_type=jnp.float32)
    hT_ref[...] = hT2
    as_ref[...] = jnp.dot(att_s_ref[...], hT2, preferred_element_type=jnp.float32)
    ad_ref[...] = jnp.dot(att_d_ref[...], hT2, preferred_element_type=jnp.float32)


def _norm_proj(acc, den, b, WT, att_s, att_d):
    return pl.pallas_call(
        _norm_proj_body,
        grid=(NP // BLK,),
        in_specs=[
            pl.BlockSpec((2, DH, BLK), lambda j: (0, 0, j)),
            pl.BlockSpec((2, BLK), lambda j: (0, j)),
            pl.BlockSpec((DH, 1), lambda j: (0, 0)),
            pl.BlockSpec((DH, DH), lambda j: (0, 0)),
            pl.BlockSpec((1, DH), lambda j: (0, 0)),
            pl.BlockSpec((1, DH), lambda j: (0, 0)),
        ],
        out_specs=[
            pl.BlockSpec((DH, BLK), lambda j: (0, j)),
            pl.BlockSpec((1, BLK), lambda j: (0, j)),
            pl.BlockSpec((1, BLK), lambda j: (0, j)),
        ],
        out_shape=[
            jax.ShapeDtypeStruct((DH, NP), jnp.float32),
            jax.ShapeDtypeStruct((1, NP), jnp.float32),
            jax.ShapeDtypeStruct((1, NP), jnp.float32),
        ],
    )(acc, den, b, WT, att_s, att_d)


def _final_body(acc_ref, den_ref, b_ref, Wf_ref, bf_ref, out_ref):
    accs = acc_ref[0] + acc_ref[1]
    den = jnp.sum(den_ref[...], axis=0, keepdims=True)
    h = jnp.maximum(accs / (den + 1e-16) + b_ref[...], 0.0)
    out_ref[...] = lax.dot_general(
        h, Wf_ref[...], (((0,), (0,)), ((), ())),
        preferred_element_type=jnp.float32) + bf_ref[...]


def _final(acc, den, b, Wf, bf):
    return pl.pallas_call(
        _final_body,
        grid=(pl.cdiv(N, RBLK),),
        in_specs=[
            pl.BlockSpec((2, DH, RBLK), lambda j: (0, 0, j)),
            pl.BlockSpec((2, RBLK), lambda j: (0, j)),
            pl.BlockSpec((DH, 1), lambda j: (0, 0)),
            pl.BlockSpec((DH, DH), lambda j: (0, 0)),
            pl.BlockSpec((1, DH), lambda j: (0, 0)),
        ],
        out_specs=pl.BlockSpec((RBLK, DH), lambda j: (j, 0)),
        out_shape=jax.ShapeDtypeStruct((N, DH), jnp.float32),
    )(acc, den, b, Wf, bf)


def kernel(x, edge_index, W1, att_src1, att_dst1, b1,
           W2, att_src2, att_dst2, b2, Wf, bf):
    n_edges = edge_index.shape[1]
    loop = jnp.arange(N, dtype=jnp.int32)
    npad = E_PAD - (n_edges + N)
    src = jnp.concatenate(
        [edge_index[0].astype(jnp.int32), loop, jnp.zeros((npad,), jnp.int32)])
    dst = jnp.concatenate(
        [edge_index[1].astype(jnp.int32), loop, jnp.full((npad,), N, jnp.int32)])

    hT1, as1, ad1 = _proj(x, W1, att_src1.reshape(1, DH), att_dst1.reshape(1, DH))
    den1, acc1 = _gat_aggregate(src, dst, as1, ad1, hT1)

    hT2, as2, ad2 = _norm_proj(acc1, den1, b1.reshape(DH, 1), W2.T,
                               att_src2.reshape(1, DH), att_dst2.reshape(1, DH))
    den2, acc2 = _gat_aggregate(src, dst, as2, ad2, hT2)

    return _final(acc2, den2, b2.reshape(DH, 1), Wf, bf.reshape(1, DH))


# speedup vs baseline: 80.8453x; 1.1208x over previous
import functools

import jax
import jax.numpy as jnp
from jax import lax
from jax.experimental import pallas as pl
from jax.experimental.pallas import tpu as pltpu
from jax.experimental.pallas import tpu_sc as plsc

N = 10000
NP = 10240
D_IN = 128
DH = 32
CHUNK = 3200
NCHUNK = 52
E_HALF = CHUNK * NCHUNK
E_PAD = 2 * E_HALF
V_ITERS = CHUNK // 64
NZV = NP // 16
BLK = 2048
RBLK = 2048

_mesh = plsc.VectorSubcoreMesh(core_axis_name="c", subcore_axis_name="s")


@functools.partial(
    pl.kernel,
    out_type=(
        jax.ShapeDtypeStruct((2, NP), jnp.float32),
        jax.ShapeDtypeStruct((2, DH, NP), jnp.float32),
    ),
    mesh=_mesh,
    compiler_params=pltpu.CompilerParams(needs_layout_passes=False),
    scratch_types=[
        pltpu.VMEM((NP,), jnp.float32),
        pltpu.VMEM((NP,), jnp.float32),
        pltpu.VMEM((NP,), jnp.float32),
        pltpu.VMEM((NP,), jnp.float32),
        pltpu.VMEM((NP,), jnp.float32),
        pltpu.VMEM((NP,), jnp.float32),
        pltpu.VMEM((NP,), jnp.float32),
        pltpu.VMEM((2 * CHUNK,), jnp.int32),
        pltpu.VMEM((2 * CHUNK,), jnp.int32),
        pltpu.SemaphoreType.DMA,
        pltpu.SemaphoreType.DMA,
    ],
)
def _gat_aggregate(src_hbm, dst_hbm, a_s_hbm, a_d_hbm, hT_hbm,
                   denom_out, acc_out,
                   tabAS, tabAD, tabH0, tabH1, acc0, acc1, denom,
                   srcb, dstb, sem0, sem1):
    c = lax.axis_index("c")
    s = lax.axis_index("s")
    L = hT_hbm.shape[1]

    cp0 = pltpu.async_copy(a_s_hbm.at[0], tabAS.at[pl.ds(0, L)], sem0)
    cp1 = pltpu.async_copy(a_d_hbm.at[0], tabAD.at[pl.ds(0, L)], sem0)
    cp2 = pltpu.async_copy(hT_hbm.at[s], tabH0.at[pl.ds(0, L)], sem0)
    cp3 = pltpu.async_copy(hT_hbm.at[s + 16], tabH1.at[pl.ds(0, L)], sem0)

    zeros = jnp.zeros((16,), jnp.float32)

    def zero_body(i, carry):
        sl = pl.ds(i * 16, 16)
        acc0[sl] = zeros
        acc1[sl] = zeros
        denom[sl] = zeros
        return carry

    lax.fori_loop(0, NZV, zero_body, 0)

    cp0.wait()
    cp1.wait()
    cp2.wait()
    cp3.wait()

    base = c * E_HALF
    sems = (sem0, sem1)

    def _issue(g, b):
        off = base + g * CHUNK
        slot = pl.ds(b * CHUNK, CHUNK)
        pltpu.async_copy(src_hbm.at[pl.ds(off, CHUNK)], srcb.at[slot], sems[b])
        pltpu.async_copy(dst_hbm.at[pl.ds(off, CHUNK)], dstb.at[slot], sems[b])

    def _wait(g, b):
        off = base + g * CHUNK
        slot = pl.ds(b * CHUNK, CHUNK)
        pltpu.make_async_copy(src_hbm.at[pl.ds(off, CHUNK)], srcb.at[slot], sems[b]).wait()
        pltpu.make_async_copy(dst_hbm.at[pl.ds(off, CHUNK)], dstb.at[slot], sems[b]).wait()

    def _compute(b):
        sbase = b * CHUNK

        @plsc.parallel_loop(0, CHUNK // 16, unroll=4)
        def _body(i):
            sl = pl.ds(sbase + i * 16, 16)
            sv = srcb[sl]
            dv = dstb[sl]
            ga = plsc.load_gather(tabAS, [sv])
            gd = plsc.load_gather(tabAD, [dv])
            h0 = plsc.load_gather(tabH0, [sv])
            h1 = plsc.load_gather(tabH1, [sv])
            e = ga + gd
            e = jnp.where(e >= 0.0, e, e * 0.2)
            w = jnp.exp(e)
            plsc.addupdate_scatter(denom, [dv], w)
            plsc.addupdate_scatter(acc0, [dv], h0 * w)
            plsc.addupdate_scatter(acc1, [dv], h1 * w)

    _issue(0, 0)
    _issue(1, 1)

    def mbody(g2, carry):
        g = g2 * 2
        _wait(g, 0)
        _compute(0)
        _issue(g + 2, 0)
        _wait(g + 1, 1)
        _compute(1)
        _issue(g + 3, 1)
        return carry

    lax.fori_loop(0, NCHUNK // 2 - 1, mbody, 0)

    _wait(NCHUNK - 2, 0)
    _compute(0)
    _wait(NCHUNK - 1, 1)
    _compute(1)

    pltpu.sync_copy(acc0, acc_out.at[c, s])
    pltpu.sync_copy(acc1, acc_out.at[c, s + 16])

    @pl.when(s == 0)
    def _():
        pltpu.sync_copy(denom, denom_out.at[c])


def _proj_body(x_ref, W_ref, att_s_ref, att_d_ref, hT_ref, as_ref, ad_ref):
    hT = lax.dot_general(W_ref[...], x_ref[...], (((0,), (1,)), ((), ())),
                         preferred_element_type=jnp.float32)
    hT_ref[...] = hT
    as_ref[...] = jnp.dot(att_s_ref[...], hT, preferred_element_type=jnp.float32)
    ad_ref[...] = jnp.dot(att_d_ref[...], hT, preferred_element_type=jnp.float32)


def _proj(x, W, att_s, att_d):
    return pl.pallas_call(
        _proj_body,
        grid=(pl.cdiv(N, RBLK),),
        in_specs=[
            pl.BlockSpec((RBLK, D_IN), lambda j: (j, 0)),
            pl.BlockSpec((D_IN, DH), lambda j: (0, 0)),
            pl.BlockSpec((1, DH), lambda j: (0, 0)),
            pl.BlockSpec((1, DH), lambda j: (0, 0)),
        ],
        out_specs=[
            pl.BlockSpec((DH, RBLK), lambda j: (0, j)),
            pl.BlockSpec((1, RBLK), lambda j: (0, j)),
            pl.BlockSpec((1, RBLK), lambda j: (0, j)),
        ],
        out_shape=[
            jax.ShapeDtypeStruct((DH, NP), jnp.float32),
            jax.ShapeDtypeStruct((1, NP), jnp.float32),
            jax.ShapeDtypeStruct((1, NP), jnp.float32),
        ],
    )(x, W, att_s, att_d)


def _norm_proj_body(acc_ref, den_ref, b_ref, WT_ref, att_s_ref, att_d_ref,
                    hT_ref, as_ref, ad_ref):
    accs = acc_ref[0] + acc_ref[1]
    den = jnp.sum(den_ref[...], axis=0, keepdims=True)
    h = jnp.maximum(accs / (den + 1e-16) + b_ref[...], 0.0)
    hT2 = jnp.dot(WT_ref[...], h, preferred_element_type=jnp.float32)
    hT_ref[...] = hT2
    as_ref[...] = jnp.dot(att_s_ref[...], hT2, preferred_element_type=jnp.float32)
    ad_ref[...] = jnp.dot(att_d_ref[...], hT2, preferred_element_type=jnp.float32)


def _norm_proj(acc, den, b, WT, att_s, att_d):
    return pl.pallas_call(
        _norm_proj_body,
        grid=(NP // BLK,),
        in_specs=[
            pl.BlockSpec((2, DH, BLK), lambda j: (0, 0, j)),
            pl.BlockSpec((2, BLK), lambda j: (0, j)),
            pl.BlockSpec((DH, 1), lambda j: (0, 0)),
            pl.BlockSpec((DH, DH), lambda j: (0, 0)),
            pl.BlockSpec((1, DH), lambda j: (0, 0)),
            pl.BlockSpec((1, DH), lambda j: (0, 0)),
        ],
        out_specs=[
            pl.BlockSpec((DH, BLK), lambda j: (0, j)),
            pl.BlockSpec((1, BLK), lambda j: (0, j)),
            pl.BlockSpec((1, BLK), lambda j: (0, j)),
        ],
        out_shape=[
            jax.ShapeDtypeStruct((DH, NP), jnp.float32),
            jax.ShapeDtypeStruct((1, NP), jnp.float32),
            jax.ShapeDtypeStruct((1, NP), jnp.float32),
        ],
    )(acc, den, b, WT, att_s, att_d)


def _final_body(acc_ref, den_ref, b_ref, Wf_ref, bf_ref, out_ref):
    accs = acc_ref[0] + acc_ref[1]
    den = jnp.sum(den_ref[...], axis=0, keepdims=True)
    h = jnp.maximum(accs / (den + 1e-16) + b_ref[...], 0.0)
    out_ref[...] = lax.dot_general(
        h, Wf_ref[...], (((0,), (0,)), ((), ())),
        preferred_element_type=jnp.float32) + bf_ref[...]


def _final(acc, den, b, Wf, bf):
    return pl.pallas_call(
        _final_body,
        grid=(pl.cdiv(N, RBLK),),
        in_specs=[
            pl.BlockSpec((2, DH, RBLK), lambda j: (0, 0, j)),
            pl.BlockSpec((2, RBLK), lambda j: (0, j)),
            pl.BlockSpec((DH, 1), lambda j: (0, 0)),
            pl.BlockSpec((DH, DH), lambda j: (0, 0)),
            pl.BlockSpec((1, DH), lambda j: (0, 0)),
        ],
        out_specs=pl.BlockSpec((RBLK, DH), lambda j: (j, 0)),
        out_shape=jax.ShapeDtypeStruct((N, DH), jnp.float32),
    )(acc, den, b, Wf, bf)


def kernel(x, edge_index, W1, att_src1, att_dst1, b1,
           W2, att_src2, att_dst2, b2, Wf, bf):
    n_edges = edge_index.shape[1]
    loop = jnp.arange(N, dtype=jnp.int32)
    npad = E_PAD - (n_edges + N)
    src = jnp.concatenate(
        [edge_index[0].astype(jnp.int32), loop, jnp.zeros((npad,), jnp.int32)])
    dst = jnp.concatenate(
        [edge_index[1].astype(jnp.int32), loop, jnp.full((npad,), N, jnp.int32)])

    hT1, as1, ad1 = _proj(x, W1, att_src1.reshape(1, DH), att_dst1.reshape(1, DH))
    den1, acc1 = _gat_aggregate(src, dst, as1, ad1, hT1)

    hT2, as2, ad2 = _norm_proj(acc1, den1, b1.reshape(DH, 1), W2.T,
                               att_src2.reshape(1, DH), att_dst2.reshape(1, DH))
    den2, acc2 = _gat_aggregate(src, dst, as2, ad2, hT2)

    return _final(acc2, den2, b2.reshape(DH, 1), Wf, bf.reshape(1, DH))


# speedup vs baseline: 96.7378x; 1.1966x over previous
import functools

import jax
import jax.numpy as jnp
from jax import lax
from jax.experimental import pallas as pl
from jax.experimental.pallas import tpu as pltpu
from jax.experimental.pallas import tpu_sc as plsc

N = 10000
NP = 10240
D_IN = 128
DH = 32
E = 320000
CHUNK = 3200
E_HALF = E // 2
NCHUNK = E_HALF // CHUNK
NZV = NP // 16
BLK = 2048
RBLK = 2048

_mesh = plsc.VectorSubcoreMesh(core_axis_name="c", subcore_axis_name="s")


@functools.partial(
    pl.kernel,
    out_type=(
        jax.ShapeDtypeStruct((2, NP), jnp.float32),
        jax.ShapeDtypeStruct((2, DH, NP), jnp.float32),
    ),
    mesh=_mesh,
    compiler_params=pltpu.CompilerParams(needs_layout_passes=False),
    scratch_types=[
        pltpu.VMEM((NP,), jnp.float32),
        pltpu.VMEM((NP,), jnp.float32),
        pltpu.VMEM((NP,), jnp.float32),
        pltpu.VMEM((NP,), jnp.float32),
        pltpu.VMEM((NP,), jnp.float32),
        pltpu.VMEM((NP,), jnp.float32),
        pltpu.VMEM((NP,), jnp.float32),
        pltpu.VMEM((2 * CHUNK,), jnp.int32),
        pltpu.VMEM((2 * CHUNK,), jnp.int32),
        pltpu.SemaphoreType.DMA,
        pltpu.SemaphoreType.DMA,
    ],
)
def _gat_aggregate(ei_hbm, a_s_hbm, a_d_hbm, hT_hbm,
                   denom_out, acc_out,
                   tabAS, tabAD, tabH0, tabH1, acc0, acc1, denom,
                   srcb, dstb, sem0, sem1):
    c = lax.axis_index("c")
    s = lax.axis_index("s")
    L = hT_hbm.shape[1]

    cp0 = pltpu.async_copy(a_s_hbm.at[0], tabAS.at[pl.ds(0, L)], sem0)
    cp1 = pltpu.async_copy(a_d_hbm.at[0], tabAD.at[pl.ds(0, L)], sem0)
    cp2 = pltpu.async_copy(hT_hbm.at[s], tabH0.at[pl.ds(0, L)], sem0)
    cp3 = pltpu.async_copy(hT_hbm.at[s + 16], tabH1.at[pl.ds(0, L)], sem0)

    zeros = jnp.zeros((16,), jnp.float32)

    @pl.when(c == 1)
    def _zero():
        @plsc.parallel_loop(0, NZV, unroll=4)
        def _zbody(i):
            sl = pl.ds(i * 16, 16)
            acc0[sl] = zeros
            acc1[sl] = zeros
            denom[sl] = zeros

    cp0.wait()
    cp1.wait()
    cp2.wait()
    cp3.wait()

    @pl.when(c == 0)
    def _selfpass():
        @plsc.parallel_loop(0, N // 16, unroll=4)
        def _sbody(i):
            sl = pl.ds(i * 16, 16)
            e = tabAS[sl] + tabAD[sl]
            e = jnp.where(e >= 0.0, e, e * 0.2)
            w = jnp.exp(e)
            denom[sl] = w
            acc0[sl] = w * tabH0[sl]
            acc1[sl] = w * tabH1[sl]

    base = c * E_HALF
    sems = (sem0, sem1)

    def _issue(g, b):
        off = base + g * CHUNK
        slot = pl.ds(b * CHUNK, CHUNK)
        pltpu.async_copy(ei_hbm.at[0, pl.ds(off, CHUNK)], srcb.at[slot], sems[b])
        pltpu.async_copy(ei_hbm.at[1, pl.ds(off, CHUNK)], dstb.at[slot], sems[b])

    def _wait(g, b):
        off = base + g * CHUNK
        slot = pl.ds(b * CHUNK, CHUNK)
        pltpu.make_async_copy(ei_hbm.at[0, pl.ds(off, CHUNK)], srcb.at[slot], sems[b]).wait()
        pltpu.make_async_copy(ei_hbm.at[1, pl.ds(off, CHUNK)], dstb.at[slot], sems[b]).wait()

    def _compute(b):
        sbase = b * CHUNK

        @plsc.parallel_loop(0, CHUNK // 16, unroll=4)
        def _body(i):
            sl = pl.ds(sbase + i * 16, 16)
            sv = srcb[sl]
            dv = dstb[sl]
            ga = plsc.load_gather(tabAS, [sv])
            gd = plsc.load_gather(tabAD, [dv])
            h0 = plsc.load_gather(tabH0, [sv])
            h1 = plsc.load_gather(tabH1, [sv])
            e = ga + gd
            e = jnp.where(e >= 0.0, e, e * 0.2)
            w = jnp.exp(e)
            plsc.addupdate_scatter(denom, [dv], w)
            plsc.addupdate_scatter(acc0, [dv], h0 * w)
            plsc.addupdate_scatter(acc1, [dv], h1 * w)

    _issue(0, 0)
    _issue(1, 1)

    def mbody(g2, carry):
        g = g2 * 2
        _wait(g, 0)
        _compute(0)
        _issue(g + 2, 0)
        _wait(g + 1, 1)
        _compute(1)
        _issue(g + 3, 1)
        return carry

    lax.fori_loop(0, NCHUNK // 2 - 1, mbody, 0)

    _wait(NCHUNK - 2, 0)
    _compute(0)
    _wait(NCHUNK - 1, 1)
    _compute(1)

    pltpu.sync_copy(acc0, acc_out.at[c, s])
    pltpu.sync_copy(acc1, acc_out.at[c, s + 16])

    @pl.when(s == 0)
    def _():
        pltpu.sync_copy(denom, denom_out.at[c])


def _proj_body(x_ref, W_ref, att_s_ref, att_d_ref, hT_ref, as_ref, ad_ref):
    hT = lax.dot_general(W_ref[...], x_ref[...], (((0,), (1,)), ((), ())),
                         preferred_element_type=jnp.float32)
    hT_ref[...] = hT
    as_ref[...] = jnp.dot(att_s_ref[...], hT, preferred_element_type=jnp.float32)
    ad_ref[...] = jnp.dot(att_d_ref[...], hT, preferred_element_type=jnp.float32)


def _proj(x, W, att_s, att_d):
    return pl.pallas_call(
        _proj_body,
        grid=(pl.cdiv(N, RBLK),),
        in_specs=[
            pl.BlockSpec((RBLK, D_IN), lambda j: (j, 0)),
            pl.BlockSpec((D_IN, DH), lambda j: (0, 0)),
            pl.BlockSpec((1, DH), lambda j: (0, 0)),
            pl.BlockSpec((1, DH), lambda j: (0, 0)),
        ],
        out_specs=[
            pl.BlockSpec((DH, RBLK), lambda j: (0, j)),
            pl.BlockSpec((1, RBLK), lambda j: (0, j)),
            pl.BlockSpec((1, RBLK), lambda j: (0, j)),
        ],
        out_shape=[
            jax.ShapeDtypeStruct((DH, NP), jnp.float32),
            jax.ShapeDtypeStruct((1, NP), jnp.float32),
            jax.ShapeDtypeStruct((1, NP), jnp.float32),
        ],
    )(x, W, att_s, att_d)


def _norm_proj_body(acc_ref, den_ref, b_ref, WT_ref, att_s_ref, att_d_ref,
                    hT_ref, as_ref, ad_ref):
    accs = acc_ref[0] + acc_ref[1]
    den = jnp.sum(den_ref[...], axis=0, keepdims=True)
    h = jnp.maximum(accs / (den + 1e-16) + b_ref[...], 0.0)
    hT2 = jnp.dot(WT_ref[...], h, preferred_element_type=jnp.float32)
    hT_ref[...] = hT2
    as_ref[...] = jnp.dot(att_s_ref[...], hT2, preferred_element_type=jnp.float32)
    ad_ref[...] = jnp.dot(att_d_ref[...], hT2, preferred_element_type=jnp.float32)


def _norm_proj(acc, den, b, WT, att_s, att_d):
    return pl.pallas_call(
        _norm_proj_body,
        grid=(NP // BLK,),
        in_specs=[
            pl.BlockSpec((2, DH, BLK), lambda j: (0, 0, j)),
            pl.BlockSpec((2, BLK), lambda j: (0, j)),
            pl.BlockSpec((DH, 1), lambda j: (0, 0)),
            pl.BlockSpec((DH, DH), lambda j: (0, 0)),
            pl.BlockSpec((1, DH), lambda j: (0, 0)),
            pl.BlockSpec((1, DH), lambda j: (0, 0)),
        ],
        out_specs=[
            pl.BlockSpec((DH, BLK), lambda j: (0, j)),
            pl.BlockSpec((1, BLK), lambda j: (0, j)),
            pl.BlockSpec((1, BLK), lambda j: (0, j)),
        ],
        out_shape=[
            jax.ShapeDtypeStruct((DH, NP), jnp.float32),
            jax.ShapeDtypeStruct((1, NP), jnp.float32),
            jax.ShapeDtypeStruct((1, NP), jnp.float32),
        ],
    )(acc, den, b, WT, att_s, att_d)


def _final_body(acc_ref, den_ref, b_ref, Wf_ref, bf_ref, out_ref):
    accs = acc_ref[0] + acc_ref[1]
    den = jnp.sum(den_ref[...], axis=0, keepdims=True)
    h = jnp.maximum(accs / (den + 1e-16) + b_ref[...], 0.0)
    out_ref[...] = lax.dot_general(
        h, Wf_ref[...], (((0,), (0,)), ((), ())),
        preferred_element_type=jnp.float32) + bf_ref[...]


def _final(acc, den, b, Wf, bf):
    return pl.pallas_call(
        _final_body,
        grid=(pl.cdiv(N, RBLK),),
        in_specs=[
            pl.BlockSpec((2, DH, RBLK), lambda j: (0, 0, j)),
            pl.BlockSpec((2, RBLK), lambda j: (0, j)),
            pl.BlockSpec((DH, 1), lambda j: (0, 0)),
            pl.BlockSpec((DH, DH), lambda j: (0, 0)),
            pl.BlockSpec((1, DH), lambda j: (0, 0)),
        ],
        out_specs=pl.BlockSpec((RBLK, DH), lambda j: (j, 0)),
        out_shape=jax.ShapeDtypeStruct((N, DH), jnp.float32),
    )(acc, den, b, Wf, bf)


def kernel(x, edge_index, W1, att_src1, att_dst1, b1,
           W2, att_src2, att_dst2, b2, Wf, bf):
    hT1, as1, ad1 = _proj(x, W1, att_src1.reshape(1, DH), att_dst1.reshape(1, DH))
    den1, acc1 = _gat_aggregate(edge_index, as1, ad1, hT1)

    hT2, as2, ad2 = _norm_proj(acc1, den1, b1.reshape(DH, 1), W2.T,
                               att_src2.reshape(1, DH), att_dst2.reshape(1, DH))
    den2, acc2 = _gat_aggregate(edge_index, as2, ad2, hT2)

    return _final(acc2, den2, b2.reshape(DH, 1), Wf, bf.reshape(1, DH))
